# baseline (speedup 1.0000x reference)
import functools
import math

import jax
import jax.numpy as jnp
import numpy as np
from jax import lax
from jax.experimental import pallas as pl
from jax.experimental.pallas import tpu as pltpu
from jax.experimental.pallas import tpu_sc as plsc

VOCAB = 100000
DIM = 1024
SEQ = 2048
BATCH = 4
TOKENS = BATCH * SEQ
LN_EPS = 1e-12
EPS_SCALED = LN_EPS / 1024.0

NUM_WORKERS = 32
TOK_PER_W = TOKENS // NUM_WORKERS
CH = 32
NCH = TOK_PER_W // CH
NV = DIM // 16


@functools.lru_cache(maxsize=1)
def _pe_div32() -> np.ndarray:
    position = np.arange(0, SEQ, dtype=np.float64)[:, None]
    div_term = np.exp(np.arange(0, DIM, 2, dtype=np.float64) * -(math.log(10000.0) / DIM))
    pe = np.zeros((SEQ, DIM), dtype=np.float32)
    pe[:, 0::2] = np.sin(position * div_term).astype(np.float32)
    pe[:, 1::2] = np.cos(position * div_term).astype(np.float32)
    return pe / np.float32(math.sqrt(DIM))


def _lane_sum(red_v, vec):
    red_v[pl.ds(0, 16)] = vec
    for off in (8, 4, 2, 1):
        a = red_v[pl.ds(0, 16)]
        b = red_v[pl.ds(off, 16)]
        red_v[pl.ds(0, 16)] = a + b
    return red_v[pl.ds(0, 16)][0]


def _sc_body(ids_hbm, tts_hbm, word_hbm, type_hbm, gam_hbm, bet_hbm, pe_hbm,
             out_hbm, idx_v, tts_v, wbuf, pe_v, type_v, gam_v, bet_v, red_v, sem):
    wid = lax.axis_index("s") * 2 + lax.axis_index("c")
    base = wid * TOK_PER_W

    pltpu.sync_copy(type_hbm, type_v)
    pltpu.sync_copy(gam_hbm, gam_v)
    pltpu.sync_copy(bet_hbm, bet_v)

    def chunk_body(c, carry):
        g0 = base + c * CH
        s0 = lax.rem(g0, SEQ)
        pltpu.sync_copy(ids_hbm.at[pl.ds(g0, CH)], idx_v)
        pltpu.sync_copy(tts_hbm.at[pl.ds(g0, CH)], tts_v.at[pl.ds(0, CH)])
        pltpu.sync_copy(pe_hbm.at[pl.ds(s0, CH)], pe_v)
        pltpu.async_copy(word_hbm.at[idx_v], wbuf, sem).wait()

        def token_body(j, inner):
            toff = tts_v[pl.ds(j, 16)][0] * DIM
            s1 = jnp.zeros((16,), jnp.float32)
            s2 = jnp.zeros((16,), jnp.float32)
            for k in range(NV):
                w = wbuf[j, pl.ds(k * 16, 16)]
                p = pe_v[j, pl.ds(k * 16, 16)]
                t = type_v[pl.ds(toff + k * 16, 16)]
                x = w + p + t
                wbuf[j, pl.ds(k * 16, 16)] = x
                s1 = s1 + x
                s2 = s2 + x * x
            m = _lane_sum(red_v, s1) * (1.0 / DIM)
            var = _lane_sum(red_v, s2) * (1.0 / DIM) - m * m
            vs = var + EPS_SCALED
            seed_i = jnp.int32(0x5F3759DF) - lax.shift_right_logical(
                lax.bitcast_convert_type(vs, jnp.int32), 1)
            ys = lax.bitcast_convert_type(seed_i, jnp.float32)
            for _ in range(3):
                ys = ys * (1.5 - 0.5 * vs * ys * ys)
            y = jnp.full((16,), ys, jnp.float32)
            mv = jnp.full((16,), m, jnp.float32)
            for k in range(NV):
                x = wbuf[j, pl.ds(k * 16, 16)]
                g = gam_v[pl.ds(k * 16, 16)]
                b = bet_v[pl.ds(k * 16, 16)]
                wbuf[j, pl.ds(k * 16, 16)] = (x - mv) * y * g + b
            return inner

        lax.fori_loop(0, CH, token_body, None)
        pltpu.sync_copy(wbuf, out_hbm.at[pl.ds(g0, CH)])
        return carry

    lax.fori_loop(0, NCH, chunk_body, None)


def kernel(input_ids, token_type_ids, word_table, type_table, ln_gamma, ln_beta):
    ids_flat = input_ids.reshape(TOKENS).astype(jnp.int32)
    tts_flat = token_type_ids.reshape(TOKENS).astype(jnp.int32)
    type_flat = type_table.reshape(2 * DIM)
    pe32 = jnp.asarray(_pe_div32())

    mesh = plsc.VectorSubcoreMesh(core_axis_name="c", subcore_axis_name="s")
    run = functools.partial(
        pl.kernel,
        mesh=mesh,
        out_type=jax.ShapeDtypeStruct((TOKENS, DIM), jnp.float32),
        scratch_types=[
            pltpu.VMEM((CH,), jnp.int32),
            pltpu.VMEM((CH + 16,), jnp.int32),
            pltpu.VMEM((CH, DIM), jnp.float32),
            pltpu.VMEM((CH, DIM), jnp.float32),
            pltpu.VMEM((2 * DIM,), jnp.float32),
            pltpu.VMEM((DIM,), jnp.float32),
            pltpu.VMEM((DIM,), jnp.float32),
            pltpu.VMEM((32,), jnp.float32),
            pltpu.SemaphoreType.DMA,
        ],
    )(_sc_body)
    out = run(ids_flat, tts_flat, word_table, type_flat, ln_gamma, ln_beta, pe32)
    return out.reshape(BATCH, SEQ, DIM)


# speedup vs baseline: 3.1593x; 3.1593x over previous
import functools
import math

import jax
import jax.numpy as jnp
import numpy as np
from jax import lax
from jax.experimental import pallas as pl
from jax.experimental.pallas import tpu as pltpu
from jax.experimental.pallas import tpu_sc as plsc

VOCAB = 100000
DIM = 1024
SEQ = 2048
BATCH = 4
TOKENS = BATCH * SEQ
LN_EPS = 1e-12
EPS_SCALED = LN_EPS / 1024.0

NUM_WORKERS = 32
TOK_PER_W = TOKENS // NUM_WORKERS
CH = 32
NCH = TOK_PER_W // CH
TB = 256
SBLK = SEQ // TB


@functools.lru_cache(maxsize=1)
def _pe_div32() -> np.ndarray:
    position = np.arange(0, SEQ, dtype=np.float64)[:, None]
    div_term = np.exp(np.arange(0, DIM, 2, dtype=np.float64) * -(math.log(10000.0) / DIM))
    pe = np.zeros((SEQ, DIM), dtype=np.float32)
    pe[:, 0::2] = np.sin(position * div_term).astype(np.float32)
    pe[:, 1::2] = np.cos(position * div_term).astype(np.float32)
    return pe / np.float32(math.sqrt(DIM))


def _sc_gather_body(ids_hbm, word_hbm, out_hbm, idx_v, wbuf0, wbuf1,
                    gs0, gs1, ws0, ws1):
    wid = lax.axis_index("s") * 2 + lax.axis_index("c")
    base = wid * TOK_PER_W
    pltpu.sync_copy(ids_hbm.at[pl.ds(base, TOK_PER_W)], idx_v)

    bufs = (wbuf0, wbuf1)
    gsems = (gs0, gs1)
    wsems = (ws0, ws1)

    def start_gather(c):
        p = c % 2
        return pltpu.async_copy(
            word_hbm.at[idx_v.at[pl.ds(c * CH, CH)]], bufs[p], gsems[p])

    gets = [None] * NCH
    writes = [None] * NCH
    gets[0] = start_gather(0)
    gets[1] = start_gather(1)
    for c in range(NCH):
        p = c % 2
        gets[c].wait()
        writes[c] = pltpu.async_copy(
            bufs[p], out_hbm.at[pl.ds(base + c * CH, CH)], wsems[p])
        if c + 2 < NCH:
            writes[c].wait()
            gets[c + 2] = start_gather(c + 2)
    writes[-2].wait()
    writes[-1].wait()


def _sc_gather(ids_flat, word_table):
    mesh = plsc.VectorSubcoreMesh(core_axis_name="c", subcore_axis_name="s")
    run = functools.partial(
        pl.kernel,
        mesh=mesh,
        out_type=jax.ShapeDtypeStruct((TOKENS, DIM), jnp.float32),
        scratch_types=[
            pltpu.VMEM((TOK_PER_W,), jnp.int32),
            pltpu.VMEM((CH, DIM), jnp.float32),
            pltpu.VMEM((CH, DIM), jnp.float32),
            pltpu.SemaphoreType.DMA,
            pltpu.SemaphoreType.DMA,
            pltpu.SemaphoreType.DMA,
            pltpu.SemaphoreType.DMA,
        ],
    )(_sc_gather_body)
    return run(ids_flat, word_table)


def _tc_ln_body(w_ref, tt_ref, pe_ref, type_ref, g_ref, b_ref, o_ref):
    w = w_ref[...]
    pe = pe_ref[...]
    tt = tt_ref[...].astype(jnp.float32)
    t0 = type_ref[0:1, :]
    t1 = type_ref[1:2, :]
    x = w + pe + t0 + tt * (t1 - t0)
    m = jnp.mean(x, axis=1, keepdims=True)
    xc = x - m
    var = jnp.mean(xc * xc, axis=1, keepdims=True)
    y = xc * lax.rsqrt(var + EPS_SCALED)
    o_ref[...] = y * g_ref[...] + b_ref[...]


def _tc_ln(gathered, tts_col, pe32, type_table, gamma2d, beta2d):
    return pl.pallas_call(
        _tc_ln_body,
        grid=(SBLK, BATCH),
        in_specs=[
            pl.BlockSpec((TB, DIM), lambda s, b: (b * SBLK + s, 0)),
            pl.BlockSpec((TB, 1), lambda s, b: (b * SBLK + s, 0)),
            pl.BlockSpec((TB, DIM), lambda s, b: (s, 0)),
            pl.BlockSpec((2, DIM), lambda s, b: (0, 0)),
            pl.BlockSpec((1, DIM), lambda s, b: (0, 0)),
            pl.BlockSpec((1, DIM), lambda s, b: (0, 0)),
        ],
        out_specs=pl.BlockSpec((TB, DIM), lambda s, b: (b * SBLK + s, 0)),
        out_shape=jax.ShapeDtypeStruct((TOKENS, DIM), jnp.float32),
        compiler_params=pltpu.CompilerParams(
            dimension_semantics=("arbitrary", "arbitrary")),
    )(gathered, tts_col, pe32, type_table, gamma2d, beta2d)


def kernel(input_ids, token_type_ids, word_table, type_table, ln_gamma, ln_beta):
    ids_flat = input_ids.reshape(TOKENS).astype(jnp.int32)
    tts_col = token_type_ids.reshape(TOKENS, 1).astype(jnp.int32)
    pe32 = jnp.asarray(_pe_div32())

    gathered = _sc_gather(ids_flat, word_table)
    out = _tc_ln(gathered, tts_col, pe32, type_table,
                 ln_gamma.reshape(1, DIM), ln_beta.reshape(1, DIM))
    return out.reshape(BATCH, SEQ, DIM)
